# baseline (speedup 1.0000x reference)
import functools

import jax
import jax.numpy as jnp
from jax import lax
from jax.experimental import pallas as pl
from jax.experimental.pallas import tpu as pltpu
from jax.experimental.pallas import tpu_sc as plsc

VOCAB = 1000000
DIM = 16
NCLS = 6
B = 16384
NNEG = 20

NC = 2
NS = 16
NW = NC * NS
CH = 128
NBUF = 2

U_CHUNKS = B // NW // CH
N_CHUNKS = B * NNEG // NW // CH
U_IDX = B // NW
N_IDX = B * NNEG // NW


@functools.cache
def _make_sc_gather():
    mesh = plsc.VectorSubcoreMesh(
        core_axis_name="c", subcore_axis_name="s", num_cores=NC, num_subcores=NS
    )
    return functools.partial(
        pl.kernel,
        out_type=(
            jax.ShapeDtypeStruct((DIM, B), jnp.float32),
            jax.ShapeDtypeStruct((DIM, B), jnp.float32),
            jax.ShapeDtypeStruct((DIM, NNEG * B), jnp.float32),
        ),
        mesh=mesh,
        compiler_params=pltpu.CompilerParams(use_tc_tiling_on_sc=False),
        scratch_types=[
            pltpu.VMEM((U_CHUNKS, CH), jnp.int32),
            pltpu.VMEM((U_CHUNKS, CH), jnp.int32),
            pltpu.VMEM((N_CHUNKS, CH), jnp.int32),
            pltpu.VMEM((NBUF, DIM, CH), jnp.float32),
            pltpu.SemaphoreType.DMA,
            pltpu.SemaphoreType.DMA,
        ],
    )(_sc_gather_body)


def _sc_gather_body(ut_hbm, vt_hbm, pu_hbm, pv_hbm, ng_hbm, eu_hbm, ev_hbm,
                    en_hbm, idx_u, idx_v, idx_n, bufs, s0, s1):
    sems = (s0, s1)
    wid = lax.axis_index("s") * NC + lax.axis_index("c")

    pltpu.sync_copy(pu_hbm.at[wid], idx_u)
    pltpu.sync_copy(pv_hbm.at[wid], idx_v)
    pltpu.sync_copy(ng_hbm.at[wid], idx_n)

    def seg(tblT, idx2, outT, nchunks, base):
        def start(j, slot):
            for d in range(DIM):
                pltpu.async_copy(
                    tblT.at[d].at[idx2.at[j]], bufs.at[slot, d], sems[slot]
                )

        def drain(j, slot):
            for d in range(DIM):
                pltpu.make_async_copy(
                    tblT.at[d].at[idx2.at[j]], bufs.at[slot, d], sems[slot]
                ).wait()

        for slot in range(NBUF):
            start(slot, slot)

        @pl.loop(0, nchunks // NBUF)
        def _group(g):
            for slot in range(NBUF):
                j = g * NBUF + slot
                drain(j, slot)
                pltpu.sync_copy(
                    bufs.at[slot], outT.at[:, pl.ds(base + j * CH, CH)]
                )

                @pl.when(j + NBUF < nchunks)
                def _():
                    start(j + NBUF, slot)

    seg(ut_hbm, idx_u, eu_hbm, U_CHUNKS, wid * U_IDX)
    seg(vt_hbm, idx_v, ev_hbm, U_CHUNKS, wid * U_IDX)
    seg(vt_hbm, idx_n, en_hbm, N_CHUNKS, wid * N_IDX)


def _tc_body(eu_ref, ev_ref, en_ref, w_ref, b_ref, loss_ref, dur_ref):
    n = pl.program_id(0)
    eu = eu_ref[...]
    enb = en_ref[...]

    nd = jnp.sum(enb * eu, axis=0, keepdims=True)
    nd = jnp.clip(nd, -10.0, 10.0)
    part = jnp.sum(jnp.log1p(jnp.exp(nd)))

    @pl.when(n == 0)
    def _():
        ev = ev_ref[...]
        score = jnp.sum(eu * ev, axis=0, keepdims=True)
        score = jnp.clip(score, -10.0, 10.0)
        pos = jnp.sum(jnp.log1p(jnp.exp(-score)))
        loss_ref[...] = jnp.full((1, 1), pos * (1.0 / B), jnp.float32)
        dur_ref[...] = (
            jnp.dot(w_ref[...], eu, preferred_element_type=jnp.float32)
            + b_ref[...]
        )

    loss_ref[...] = loss_ref[...] + jnp.full((1, 1), part * (1.0 / B), jnp.float32)


_tc_compute = pl.pallas_call(
    _tc_body,
    grid=(NNEG,),
    in_specs=[
        pl.BlockSpec((DIM, B), lambda n: (0, 0)),
        pl.BlockSpec((DIM, B), lambda n: (0, 0)),
        pl.BlockSpec((DIM, B), lambda n: (0, n)),
        pl.BlockSpec((NCLS, DIM), lambda n: (0, 0)),
        pl.BlockSpec((NCLS, 1), lambda n: (0, 0)),
    ],
    out_specs=[
        pl.BlockSpec((1, 1), lambda n: (0, 0)),
        pl.BlockSpec((NCLS, B), lambda n: (0, 0)),
    ],
    out_shape=[
        jax.ShapeDtypeStruct((1, 1), jnp.float32),
        jax.ShapeDtypeStruct((NCLS, B), jnp.float32),
    ],
)


def kernel(u_emb, v_emb, W, b, pos_u, pos_v, neg_v):
    ut = u_emb.T
    vt = v_emb.T
    pu = pos_u.astype(jnp.int32).reshape(NW, U_CHUNKS, CH)
    pv = pos_v.astype(jnp.int32).reshape(NW, U_CHUNKS, CH)
    ng = neg_v.astype(jnp.int32).T.reshape(NW, N_CHUNKS, CH)
    eu_t, ev_t, en_t = _make_sc_gather()(ut, vt, pu, pv, ng)
    loss_arr, dur_t = _tc_compute(
        eu_t, ev_t, en_t, W, b.reshape(NCLS, 1)
    )
    return loss_arr[0, 0], dur_t.T


# speedup vs baseline: 2.7890x; 2.7890x over previous
import functools

import jax
import jax.numpy as jnp
from jax import lax
from jax.experimental import pallas as pl
from jax.experimental.pallas import tpu as pltpu
from jax.experimental.pallas import tpu_sc as plsc

VOCAB = 1000000
DIM = 16
NCLS = 6
B = 16384
NNEG = 20

NC = 2
NS = 16
NW = NC * NS
CH = 128
NBUF = 4

U_CHUNKS = B // NW // CH
N_CHUNKS = B * NNEG // NW // CH
U_IDX = B // NW
N_IDX = B * NNEG // NW


@functools.cache
def _make_sc_gather():
    mesh = plsc.VectorSubcoreMesh(
        core_axis_name="c", subcore_axis_name="s", num_cores=NC, num_subcores=NS
    )
    return functools.partial(
        pl.kernel,
        out_type=(
            jax.ShapeDtypeStruct((B, DIM), jnp.float32),
            jax.ShapeDtypeStruct((B, DIM), jnp.float32),
            jax.ShapeDtypeStruct((NNEG * B, DIM), jnp.float32),
        ),
        mesh=mesh,
        compiler_params=pltpu.CompilerParams(use_tc_tiling_on_sc=False),
        scratch_types=[
            pltpu.VMEM((U_CHUNKS, CH), jnp.int32),
            pltpu.VMEM((U_CHUNKS, CH), jnp.int32),
            pltpu.VMEM((N_CHUNKS, CH), jnp.int32),
            pltpu.VMEM((NBUF, CH, DIM), jnp.float32),
            pltpu.SemaphoreType.DMA,
            pltpu.SemaphoreType.DMA,
            pltpu.SemaphoreType.DMA,
            pltpu.SemaphoreType.DMA,
        ],
    )(_sc_gather_body)


def _sc_gather_body(u_hbm, v_hbm, pu_hbm, pv_hbm, ng_hbm, eu_hbm, ev_hbm,
                    en_hbm, idx_u, idx_v, idx_n, bufs, s0, s1, s2, s3):
    sems = (s0, s1, s2, s3)
    wid = lax.axis_index("s") * NC + lax.axis_index("c")

    pltpu.sync_copy(pu_hbm.at[wid], idx_u)
    pltpu.sync_copy(pv_hbm.at[wid], idx_v)
    pltpu.sync_copy(ng_hbm.at[wid], idx_n)

    def seg(tbl, idx2, out, nchunks, base_rows):
        for s in range(NBUF):
            pltpu.async_copy(tbl.at[idx2.at[s]], bufs.at[s], sems[s])

        @pl.loop(0, nchunks // NBUF)
        def _group(g):
            for s in range(NBUF):
                j = g * NBUF + s
                pltpu.make_async_copy(tbl.at[idx2.at[j]], bufs.at[s], sems[s]).wait()
                pltpu.sync_copy(bufs.at[s], out.at[pl.ds(base_rows + j * CH, CH)])

                @pl.when(j + NBUF < nchunks)
                def _():
                    pltpu.async_copy(tbl.at[idx2.at[j + NBUF]], bufs.at[s], sems[s])

    seg(u_hbm, idx_u, eu_hbm, U_CHUNKS, wid * U_IDX)
    seg(v_hbm, idx_v, ev_hbm, U_CHUNKS, wid * U_IDX)
    seg(v_hbm, idx_n, en_hbm, N_CHUNKS, wid * N_IDX)


def _tc_body(eu_ref, ev_ref, en_ref, w_ref, b_ref, loss_ref, dur_ref):
    n = pl.program_id(0)
    eu = eu_ref[...]
    enb = en_ref[...]

    nd = jnp.sum(enb * eu, axis=0, keepdims=True)
    nd = jnp.clip(nd, -10.0, 10.0)
    part = jnp.sum(jnp.log1p(jnp.exp(nd)))

    @pl.when(n == 0)
    def _():
        ev = ev_ref[...]
        score = jnp.sum(eu * ev, axis=0, keepdims=True)
        score = jnp.clip(score, -10.0, 10.0)
        pos = jnp.sum(jnp.log1p(jnp.exp(-score)))
        loss_ref[...] = jnp.full((1, 1), pos * (1.0 / B), jnp.float32)
        dur_ref[...] = (
            jnp.dot(w_ref[...], eu, preferred_element_type=jnp.float32)
            + b_ref[...]
        )

    loss_ref[...] = loss_ref[...] + jnp.full((1, 1), part * (1.0 / B), jnp.float32)


_tc_compute = pl.pallas_call(
    _tc_body,
    grid=(NNEG,),
    in_specs=[
        pl.BlockSpec((DIM, B), lambda n: (0, 0)),
        pl.BlockSpec((DIM, B), lambda n: (0, 0)),
        pl.BlockSpec((DIM, B), lambda n: (0, n)),
        pl.BlockSpec((NCLS, DIM), lambda n: (0, 0)),
        pl.BlockSpec((NCLS, 1), lambda n: (0, 0)),
    ],
    out_specs=[
        pl.BlockSpec((1, 1), lambda n: (0, 0)),
        pl.BlockSpec((NCLS, B), lambda n: (0, 0)),
    ],
    out_shape=[
        jax.ShapeDtypeStruct((1, 1), jnp.float32),
        jax.ShapeDtypeStruct((NCLS, B), jnp.float32),
    ],
)


def kernel(u_emb, v_emb, W, b, pos_u, pos_v, neg_v):
    pu = pos_u.astype(jnp.int32).reshape(NW, U_CHUNKS, CH)
    pv = pos_v.astype(jnp.int32).reshape(NW, U_CHUNKS, CH)
    ng = neg_v.astype(jnp.int32).T.reshape(NW, N_CHUNKS, CH)
    emb_u, emb_v, emb_neg = _make_sc_gather()(u_emb, v_emb, pu, pv, ng)
    loss_arr, dur_t = _tc_compute(
        emb_u.T, emb_v.T, emb_neg.T, W, b.reshape(NCLS, 1)
    )
    return loss_arr[0, 0], dur_t.T


# speedup vs baseline: 3.0780x; 1.1036x over previous
import functools

import jax
import jax.numpy as jnp
from jax import lax
from jax.experimental import pallas as pl
from jax.experimental.pallas import tpu as pltpu
from jax.experimental.pallas import tpu_sc as plsc

VOCAB = 1000000
DIM = 16
NCLS = 6
B = 16384
NNEG = 20

NC = 2
NS = 16
NW = NC * NS
CH = 128
NBUF = 4

U_CHUNKS = B // NW // CH
N_CHUNKS = B * NNEG // NW // CH
U_IDX = B // NW
N_IDX = B * NNEG // NW


@functools.cache
def _make_sc_gather():
    mesh = plsc.VectorSubcoreMesh(
        core_axis_name="c", subcore_axis_name="s", num_cores=NC, num_subcores=NS
    )
    return functools.partial(
        pl.kernel,
        out_type=(
            jax.ShapeDtypeStruct((DIM, B), jnp.float32),
            jax.ShapeDtypeStruct((DIM, B), jnp.float32),
            jax.ShapeDtypeStruct((DIM, NNEG * B), jnp.float32),
        ),
        mesh=mesh,
        compiler_params=pltpu.CompilerParams(use_tc_tiling_on_sc=False),
        scratch_types=[
            pltpu.VMEM((U_CHUNKS, CH), jnp.int32),
            pltpu.VMEM((U_CHUNKS, CH), jnp.int32),
            pltpu.VMEM((N_CHUNKS, CH), jnp.int32),
            pltpu.VMEM((NBUF, DIM, CH), jnp.float32),
            pltpu.SemaphoreType.DMA,
            pltpu.SemaphoreType.DMA,
            pltpu.SemaphoreType.DMA,
            pltpu.SemaphoreType.DMA,
        ],
    )(_sc_gather_body)


def _sc_gather_body(*refs):
    uds = refs[0:DIM]
    vds = refs[DIM:2 * DIM]
    pu_hbm, pv_hbm, ng_hbm = refs[2 * DIM:2 * DIM + 3]
    eu_hbm, ev_hbm, en_hbm = refs[2 * DIM + 3:2 * DIM + 6]
    idx_u, idx_v, idx_n, bufs, s0, s1, s2, s3 = refs[2 * DIM + 6:]
    sems = (s0, s1, s2, s3)
    wid = lax.axis_index("s") * NC + lax.axis_index("c")

    pltpu.sync_copy(pu_hbm.at[wid], idx_u)
    pltpu.sync_copy(pv_hbm.at[wid], idx_v)
    pltpu.sync_copy(ng_hbm.at[wid], idx_n)

    def seg(tbls, idx2, outT, nchunks, base):
        def start(j, slot):
            for d in range(DIM):
                pltpu.async_copy(
                    tbls[d].at[idx2.at[j]], bufs.at[slot, d], sems[slot]
                )

        def drain(j, slot):
            for d in range(DIM):
                pltpu.make_async_copy(
                    tbls[d].at[idx2.at[j]], bufs.at[slot, d], sems[slot]
                ).wait()

        for slot in range(NBUF):
            start(slot, slot)

        @pl.loop(0, nchunks // NBUF)
        def _group(g):
            for slot in range(NBUF):
                j = g * NBUF + slot
                drain(j, slot)
                pltpu.sync_copy(
                    bufs.at[slot], outT.at[:, pl.ds(base + j * CH, CH)]
                )

                @pl.when(j + NBUF < nchunks)
                def _():
                    start(j + NBUF, slot)

    seg(uds, idx_u, eu_hbm, U_CHUNKS, wid * U_IDX)
    seg(vds, idx_v, ev_hbm, U_CHUNKS, wid * U_IDX)
    seg(vds, idx_n, en_hbm, N_CHUNKS, wid * N_IDX)


def _tc_body(eu_ref, ev_ref, en_ref, w_ref, b_ref, loss_ref, dur_ref):
    n = pl.program_id(0)
    eu = eu_ref[...]
    enb = en_ref[...]

    nd = jnp.sum(enb * eu, axis=0, keepdims=True)
    nd = jnp.clip(nd, -10.0, 10.0)
    part = jnp.sum(jnp.log1p(jnp.exp(nd)))

    @pl.when(n == 0)
    def _():
        ev = ev_ref[...]
        score = jnp.sum(eu * ev, axis=0, keepdims=True)
        score = jnp.clip(score, -10.0, 10.0)
        pos = jnp.sum(jnp.log1p(jnp.exp(-score)))
        loss_ref[...] = jnp.full((1, 1), pos * (1.0 / B), jnp.float32)
        dur_ref[...] = (
            jnp.dot(w_ref[...], eu, preferred_element_type=jnp.float32)
            + b_ref[...]
        )

    loss_ref[...] = loss_ref[...] + jnp.full((1, 1), part * (1.0 / B), jnp.float32)


_tc_compute = pl.pallas_call(
    _tc_body,
    grid=(NNEG,),
    in_specs=[
        pl.BlockSpec((DIM, B), lambda n: (0, 0)),
        pl.BlockSpec((DIM, B), lambda n: (0, 0)),
        pl.BlockSpec((DIM, B), lambda n: (0, n)),
        pl.BlockSpec((NCLS, DIM), lambda n: (0, 0)),
        pl.BlockSpec((NCLS, 1), lambda n: (0, 0)),
    ],
    out_specs=[
        pl.BlockSpec((1, 1), lambda n: (0, 0)),
        pl.BlockSpec((NCLS, B), lambda n: (0, 0)),
    ],
    out_shape=[
        jax.ShapeDtypeStruct((1, 1), jnp.float32),
        jax.ShapeDtypeStruct((NCLS, B), jnp.float32),
    ],
)


def kernel(u_emb, v_emb, W, b, pos_u, pos_v, neg_v):
    pu = pos_u.astype(jnp.int32).reshape(NW, U_CHUNKS, CH)
    pv = pos_v.astype(jnp.int32).reshape(NW, U_CHUNKS, CH)
    ng = neg_v.astype(jnp.int32).T.reshape(NW, N_CHUNKS, CH)
    uds = tuple(u_emb[:, d] for d in range(DIM))
    vds = tuple(v_emb[:, d] for d in range(DIM))
    eu_t, ev_t, en_t = _make_sc_gather()(*uds, *vds, pu, pv, ng)
    loss_arr, dur_t = _tc_compute(eu_t, ev_t, en_t, W, b.reshape(NCLS, 1))
    return loss_arr[0, 0], dur_t.T
